# baseline (speedup 1.0000x reference)
import jax
import jax.numpy as jnp
from jax import lax
from jax.experimental import pallas as pl
from jax.experimental.pallas import tpu as pltpu
from jax.experimental.pallas import tpu_sc as plsc

N_NODES = 10000
N_EDGES = 320000
D_NODE = 128
D_EDGE = 16
HIDDEN = 32

NC = 2
NS = 16
NW = NC * NS
CH = 128
CPW = 80
PER_W = CH * CPW
E_PAD = NW * PER_W
NBUF = 4
NGRP = CPW // NBUF
ROWS_PER_TILE = N_NODES // NS

EBLK = 8192
NEBLK = E_PAD // EBLK

_slope = 0.01


def _leaky(x):
    return jnp.where(x >= 0, x, _slope * x)


def _prep_body(nodes_ref, wr_ref, ws_ref, pr_ref, ps_ref):
    n = nodes_ref[...]
    pr_ref[...] = jnp.dot(n, wr_ref[...], preferred_element_type=jnp.float32)
    ps_ref[...] = jnp.dot(n, ws_ref[...], preferred_element_type=jnp.float32)


def _edge_body(et_ref, g1_ref, g2_ref, w1_ref, b1_ref, w2_ref, b2_ref,
               out_ref, outt_ref, ps_ref):
    i = pl.program_id(0)
    x = lax.dot_general(et_ref[...], w1_ref[...], (((0,), (0,)), ((), ())),
                        preferred_element_type=jnp.float32)
    x = _leaky(x + b1_ref[...] + g1_ref[...] + g2_ref[...])
    y = jnp.dot(x, w2_ref[...], preferred_element_type=jnp.float32)
    y = _leaky(y + b2_ref[...])
    rows = lax.broadcasted_iota(jnp.int32, (EBLK, HIDDEN), 0) + i * EBLK
    y = jnp.where(rows < N_EDGES, y, 0.0)
    out_ref[...] = y
    outt_ref[...] = y.T

    @pl.when(i == 0)
    def _():
        ps_ref[...] = jnp.zeros_like(ps_ref)

    ps_ref[...] += jnp.sum(y, axis=0, keepdims=True)


def _node_body(nodes_ref, agg2_ref, psum_ref,
               wn1a_ref, wn1b_ref, bn1_ref, wn2_ref, bn2_ref,
               wg1_ref, bg1_ref, wg2_ref, bg2_ref,
               nn_ref, g_ref):
    agg = agg2_ref[0] + agg2_ref[1]
    h = jnp.dot(nodes_ref[...], wn1a_ref[...], preferred_element_type=jnp.float32)
    h = h + jnp.dot(agg, wn1b_ref[...], preferred_element_type=jnp.float32)
    h = _leaky(h + bn1_ref[...])
    nn = jnp.dot(h, wn2_ref[...], preferred_element_type=jnp.float32)
    nn = _leaky(nn + bn2_ref[...])
    nn_ref[...] = nn

    node_sum = jnp.sum(nn, axis=0, keepdims=True)
    edge_sum = psum_ref[...]
    gi = jnp.dot(node_sum, wg1_ref[0:HIDDEN, :], preferred_element_type=jnp.float32)
    gi = gi + jnp.dot(edge_sum, wg1_ref[HIDDEN:2 * HIDDEN, :],
                      preferred_element_type=jnp.float32)
    gi = _leaky(gi + bg1_ref[...])
    go = jnp.dot(gi, wg2_ref[...], preferred_element_type=jnp.float32)
    g_ref[...] = _leaky(go + bg2_ref[...])


def _sc_gather_body(pr_hbm, ps_hbm, recv2d_hbm, send2d_hbm, g1_hbm, g2_hbm,
                    idx2d, rows, gsem, wsem):
    c = lax.axis_index("c")
    s = lax.axis_index("s")
    wid = s * NC + c
    base = wid * PER_W

    for tab, idx_hbm, out_hbm in ((pr_hbm, recv2d_hbm, g1_hbm),
                                  (ps_hbm, send2d_hbm, g2_hbm)):
        pltpu.sync_copy(idx_hbm.at[pl.ds(wid * CPW, CPW)], idx2d)
        for b in range(NBUF):
            pltpu.async_copy(tab.at[idx2d.at[b]], rows.at[b], gsem.at[b])

        def group(g, _, tab=tab, out_hbm=out_hbm):
            for b in range(NBUF):
                j = g * NBUF + b
                dst = out_hbm.at[pl.ds(base + j * CH, CH)]
                pltpu.make_async_copy(tab.at[idx2d.at[b]], rows.at[b],
                                      gsem.at[b]).wait()
                pltpu.async_copy(rows.at[b], dst, wsem.at[b])
                pltpu.make_async_copy(rows.at[b], dst, wsem.at[b]).wait()
                pltpu.async_copy(tab.at[idx2d.at[NBUF + j]], rows.at[b], gsem.at[b])
            return _

        lax.fori_loop(0, NGRP - 1, group, 0)
        for b in range(NBUF):
            j = (NGRP - 1) * NBUF + b
            dst = out_hbm.at[pl.ds(base + j * CH, CH)]
            pltpu.make_async_copy(tab.at[idx2d.at[b]], rows.at[b],
                                  gsem.at[b]).wait()
            pltpu.async_copy(rows.at[b], dst, wsem.at[b])
        for b in range(NBUF):
            j = (NGRP - 1) * NBUF + b
            dst = out_hbm.at[pl.ds(base + j * CH, CH)]
            pltpu.make_async_copy(rows.at[b], dst, wsem.at[b]).wait()


def _sc_scatter_body(ne_hbm, recv2d_hbm, out_hbm,
                     shared, zbuf, idx2d, rows, lsem, ssem):
    c = lax.axis_index("c")
    s = lax.axis_index("s")
    wid = s * NC + c
    base = wid * PER_W

    def zrow(i, _):
        zbuf[i, pl.ds(0, 16)] = jnp.zeros((16,), jnp.float32)
        zbuf[i, pl.ds(16, 16)] = jnp.zeros((16,), jnp.float32)
        return _

    lax.fori_loop(0, ROWS_PER_TILE, zrow, 0)
    pltpu.sync_copy(zbuf, shared.at[pl.ds(s * ROWS_PER_TILE, ROWS_PER_TILE)])
    plsc.subcore_barrier()

    pltpu.sync_copy(recv2d_hbm.at[pl.ds(wid * CPW, CPW)], idx2d)
    for b in range(NBUF):
        pltpu.async_copy(ne_hbm.at[pl.ds(base + b * CH, CH)], rows.at[b], lsem.at[b])

    def group(g, _):
        for b in range(NBUF):
            j = g * NBUF + b
            sdst = shared.at[idx2d.at[j]]
            pltpu.make_async_copy(ne_hbm.at[pl.ds(base + j * CH, CH)],
                                  rows.at[b], lsem.at[b]).wait()
            pltpu.async_copy(rows.at[b], sdst, ssem.at[b], add=True)
            pltpu.make_async_copy(rows.at[b], sdst, ssem.at[b]).wait()
            pltpu.async_copy(ne_hbm.at[pl.ds(base + (NBUF + j) * CH, CH)],
                             rows.at[b], lsem.at[b])
        return _

    lax.fori_loop(0, NGRP - 1, group, 0)
    for b in range(NBUF):
        j = (NGRP - 1) * NBUF + b
        sdst = shared.at[idx2d.at[j]]
        pltpu.make_async_copy(ne_hbm.at[pl.ds(base + j * CH, CH)],
                              rows.at[b], lsem.at[b]).wait()
        pltpu.async_copy(rows.at[b], sdst, ssem.at[b], add=True)
        pltpu.make_async_copy(rows.at[b], sdst, ssem.at[b]).wait()
    plsc.subcore_barrier()

    pltpu.sync_copy(shared.at[pl.ds(s * ROWS_PER_TILE, ROWS_PER_TILE)], zbuf)
    pltpu.sync_copy(zbuf, out_hbm.at[c, pl.ds(s * ROWS_PER_TILE, ROWS_PER_TILE)])


def kernel(nodes, edges, globals_, senders, receivers,
           We1, be1, We2, be2, Wn1, bn1, Wn2, bn2, Wg1, bg1, Wg2, bg2):
    del globals_
    f32 = jnp.float32
    i32 = jnp.int32
    pad = jnp.zeros((E_PAD - N_EDGES,), i32)
    recv2d = jnp.concatenate([receivers.astype(i32), pad]).reshape(NW * CPW, CH)
    send2d = jnp.concatenate([senders.astype(i32), pad]).reshape(NW * CPW, CH)

    We1r = We1[D_EDGE:D_EDGE + D_NODE]
    We1s = We1[D_EDGE + D_NODE:]

    pr, ps = pl.pallas_call(
        _prep_body,
        out_shape=[jax.ShapeDtypeStruct((N_NODES, HIDDEN), f32),
                   jax.ShapeDtypeStruct((N_NODES, HIDDEN), f32)],
    )(nodes, We1r, We1s)

    mesh = plsc.VectorSubcoreMesh(core_axis_name="c", subcore_axis_name="s",
                                  num_cores=NC, num_subcores=NS)
    gather_k = pl.kernel(
        _sc_gather_body,
        out_type=[jax.ShapeDtypeStruct((E_PAD, HIDDEN), f32),
                  jax.ShapeDtypeStruct((E_PAD, HIDDEN), f32)],
        mesh=mesh,
        compiler_params=pltpu.CompilerParams(use_tc_tiling_on_sc=False),
        scratch_types=[
            pltpu.VMEM((CPW, CH), i32),
            pltpu.VMEM((NBUF, CH, HIDDEN), f32),
            pltpu.SemaphoreType.DMA((NBUF,)),
            pltpu.SemaphoreType.DMA((NBUF,)),
        ],
    )
    g1, g2 = gather_k(pr, ps, recv2d, send2d)

    edges_t = edges.T
    new_edges_pad, new_edges_t, psum = pl.pallas_call(
        _edge_body,
        grid=(NEBLK,),
        in_specs=[
            pl.BlockSpec((D_EDGE, EBLK), lambda i: (0, i)),
            pl.BlockSpec((EBLK, HIDDEN), lambda i: (i, 0)),
            pl.BlockSpec((EBLK, HIDDEN), lambda i: (i, 0)),
            pl.BlockSpec((D_EDGE, HIDDEN), lambda i: (0, 0)),
            pl.BlockSpec((1, HIDDEN), lambda i: (0, 0)),
            pl.BlockSpec((HIDDEN, HIDDEN), lambda i: (0, 0)),
            pl.BlockSpec((1, HIDDEN), lambda i: (0, 0)),
        ],
        out_specs=[
            pl.BlockSpec((EBLK, HIDDEN), lambda i: (i, 0)),
            pl.BlockSpec((HIDDEN, EBLK), lambda i: (0, i)),
            pl.BlockSpec((1, HIDDEN), lambda i: (0, 0)),
        ],
        out_shape=[jax.ShapeDtypeStruct((E_PAD, HIDDEN), f32),
                   jax.ShapeDtypeStruct((HIDDEN, N_EDGES), f32),
                   jax.ShapeDtypeStruct((1, HIDDEN), f32)],
    )(edges_t, g1, g2, We1[:D_EDGE], be1[None, :], We2, be2[None, :])
    new_edges = new_edges_t.T

    scatter_k = pl.kernel(
        _sc_scatter_body,
        out_type=jax.ShapeDtypeStruct((NC, N_NODES, HIDDEN), f32),
        mesh=mesh,
        compiler_params=pltpu.CompilerParams(use_tc_tiling_on_sc=False),
        scratch_types=[
            pltpu.VMEM_SHARED((N_NODES, HIDDEN), f32),
            pltpu.VMEM((ROWS_PER_TILE, HIDDEN), f32),
            pltpu.VMEM((CPW, CH), i32),
            pltpu.VMEM((NBUF, CH, HIDDEN), f32),
            pltpu.SemaphoreType.DMA((NBUF,)),
            pltpu.SemaphoreType.DMA((NBUF,)),
        ],
    )
    agg2 = scatter_k(new_edges_pad, recv2d)

    new_nodes, new_globals = pl.pallas_call(
        _node_body,
        out_shape=[jax.ShapeDtypeStruct((N_NODES, HIDDEN), f32),
                   jax.ShapeDtypeStruct((1, HIDDEN), f32)],
    )(nodes, agg2, psum, Wn1[:D_NODE], Wn1[D_NODE:], bn1[None, :], Wn2,
      bn2[None, :], Wg1, bg1[None, :], Wg2, bg2[None, :])

    return (new_nodes, new_edges, new_globals)


# speedup vs baseline: 1.3370x; 1.3370x over previous
import jax
import jax.numpy as jnp
from jax import lax
from jax.experimental import pallas as pl
from jax.experimental.pallas import tpu as pltpu
from jax.experimental.pallas import tpu_sc as plsc

N_NODES = 10000
N_EDGES = 320000
D_NODE = 128
D_EDGE = 16
HIDDEN = 32

NC = 2
NS = 16
NW = NC * NS
CH = 128
CPW = 80
PER_W = CH * CPW
NBUF = 4
IDXR = N_EDGES // CH
LAST_CH = (N_EDGES - (NW - 1) * PER_W) // CH
ROWS_PER_TILE = N_NODES // NS

EBLK = 6400
NEBLK = N_EDGES // EBLK

_slope = 0.01


def _leaky(x):
    return jnp.where(x >= 0, x, _slope * x)


def _prep_body(nodes_ref, wr_ref, ws_ref, pr_ref, ps_ref):
    n = nodes_ref[...]
    pr_ref[...] = jnp.dot(
        n, wr_ref[...], preferred_element_type=jnp.float32).astype(jnp.bfloat16)
    ps_ref[...] = jnp.dot(
        n, ws_ref[...], preferred_element_type=jnp.float32).astype(jnp.bfloat16)


def _edge_body(et_ref, g1_ref, g2_ref, w1_ref, b1_ref, w2_ref, b2_ref,
               out_ref, outt_ref, ps_ref):
    i = pl.program_id(0)
    x = lax.dot_general(et_ref[...], w1_ref[...], (((0,), (0,)), ((), ())),
                        preferred_element_type=jnp.float32)
    g = g1_ref[...].astype(jnp.float32) + g2_ref[...].astype(jnp.float32)
    x = _leaky(x + b1_ref[...] + g)
    y = jnp.dot(x, w2_ref[...], preferred_element_type=jnp.float32)
    y = _leaky(y + b2_ref[...])
    out_ref[...] = y
    outt_ref[...] = y.T

    @pl.when(i == 0)
    def _():
        ps_ref[...] = jnp.zeros_like(ps_ref)

    ps_ref[...] += jnp.sum(y, axis=0, keepdims=True)


def _node_body(nodes_ref, agg2_ref, psum_ref,
               wn1a_ref, wn1b_ref, bn1_ref, wn2_ref, bn2_ref,
               wg1_ref, bg1_ref, wg2_ref, bg2_ref,
               nn_ref, g_ref):
    agg = agg2_ref[0] + agg2_ref[1]
    h = jnp.dot(nodes_ref[...], wn1a_ref[...], preferred_element_type=jnp.float32)
    h = h + jnp.dot(agg, wn1b_ref[...], preferred_element_type=jnp.float32)
    h = _leaky(h + bn1_ref[...])
    nn = jnp.dot(h, wn2_ref[...], preferred_element_type=jnp.float32)
    nn = _leaky(nn + bn2_ref[...])
    nn_ref[...] = nn

    node_sum = jnp.sum(nn, axis=0, keepdims=True)
    edge_sum = psum_ref[...]
    gi = jnp.dot(node_sum, wg1_ref[0:HIDDEN, :], preferred_element_type=jnp.float32)
    gi = gi + jnp.dot(edge_sum, wg1_ref[HIDDEN:2 * HIDDEN, :],
                      preferred_element_type=jnp.float32)
    gi = _leaky(gi + bg1_ref[...])
    go = jnp.dot(gi, wg2_ref[...], preferred_element_type=jnp.float32)
    g_ref[...] = _leaky(go + bg2_ref[...])


def _sc_gather_body(pr_hbm, ps_hbm, recv2d_hbm, send2d_hbm, g1_hbm, g2_hbm,
                    tabs_sh, tbuf, idx2d, rows, gsem, wsem):
    c = lax.axis_index("c")
    s = lax.axis_index("s")
    wid = s * NC + c
    base = wid * PER_W
    nch = jnp.where(wid == NW - 1, LAST_CH, CPW)
    ngrp = nch // NBUF
    irow0 = jnp.where(wid == NW - 1, IDXR - CPW, wid * CPW)
    ioff = jnp.where(wid == NW - 1, wid * CPW - (IDXR - CPW), 0)

    for t, tab_hbm in ((0, pr_hbm), (1, ps_hbm)):
        pltpu.sync_copy(tab_hbm.at[pl.ds(s * ROWS_PER_TILE, ROWS_PER_TILE)], tbuf)
        pltpu.sync_copy(tbuf, tabs_sh.at[t, pl.ds(s * ROWS_PER_TILE, ROWS_PER_TILE)])
    plsc.subcore_barrier()

    for t, idx_hbm, out_hbm in ((0, recv2d_hbm, g1_hbm), (1, send2d_hbm, g2_hbm)):
        tab = tabs_sh.at[t]
        pltpu.sync_copy(idx_hbm.at[pl.ds(irow0, CPW)], idx2d)
        for b in range(NBUF):
            pltpu.async_copy(tab.at[idx2d.at[ioff + b]], rows.at[b], gsem.at[b])

        def group(g, _, tab=tab, out_hbm=out_hbm):
            for b in range(NBUF):
                j = g * NBUF + b
                dst = out_hbm.at[pl.ds(base + j * CH, CH)]
                pltpu.make_async_copy(tab.at[idx2d.at[ioff + b]], rows.at[b],
                                      gsem.at[b]).wait()
                pltpu.async_copy(rows.at[b], dst, wsem.at[b])
                pltpu.make_async_copy(rows.at[b], dst, wsem.at[b]).wait()
                pltpu.async_copy(tab.at[idx2d.at[ioff + NBUF + j]], rows.at[b],
                                 gsem.at[b])
            return _

        lax.fori_loop(0, ngrp - 1, group, 0)
        for b in range(NBUF):
            j = (ngrp - 1) * NBUF + b
            dst = out_hbm.at[pl.ds(base + j * CH, CH)]
            pltpu.make_async_copy(tab.at[idx2d.at[ioff + b]], rows.at[b],
                                  gsem.at[b]).wait()
            pltpu.async_copy(rows.at[b], dst, wsem.at[b])
        for b in range(NBUF):
            j = (ngrp - 1) * NBUF + b
            dst = out_hbm.at[pl.ds(base + j * CH, CH)]
            pltpu.make_async_copy(rows.at[b], dst, wsem.at[b]).wait()


def _sc_scatter_body(ne_hbm, recv2d_hbm, out_hbm,
                     shared, zbuf, idx2d, rows, lsem, ssem):
    c = lax.axis_index("c")
    s = lax.axis_index("s")
    wid = s * NC + c
    base = wid * PER_W
    nch = jnp.where(wid == NW - 1, LAST_CH, CPW)
    ngrp = nch // NBUF
    irow0 = jnp.where(wid == NW - 1, IDXR - CPW, wid * CPW)
    ioff = jnp.where(wid == NW - 1, wid * CPW - (IDXR - CPW), 0)

    def zrow(i, _):
        zbuf[i, pl.ds(0, 16)] = jnp.zeros((16,), jnp.float32)
        zbuf[i, pl.ds(16, 16)] = jnp.zeros((16,), jnp.float32)
        return _

    lax.fori_loop(0, ROWS_PER_TILE, zrow, 0)
    pltpu.sync_copy(zbuf, shared.at[pl.ds(s * ROWS_PER_TILE, ROWS_PER_TILE)])
    plsc.subcore_barrier()

    pltpu.sync_copy(recv2d_hbm.at[pl.ds(irow0, CPW)], idx2d)
    for b in range(NBUF):
        pltpu.async_copy(ne_hbm.at[pl.ds(base + b * CH, CH)], rows.at[b], lsem.at[b])

    def group(g, _):
        for b in range(NBUF):
            j = g * NBUF + b
            sdst = shared.at[idx2d.at[ioff + j]]
            pltpu.make_async_copy(ne_hbm.at[pl.ds(base + j * CH, CH)],
                                  rows.at[b], lsem.at[b]).wait()
            pltpu.async_copy(rows.at[b], sdst, ssem.at[b], add=True)
            pltpu.make_async_copy(rows.at[b], sdst, ssem.at[b]).wait()
            pltpu.async_copy(ne_hbm.at[pl.ds(base + (NBUF + j) * CH, CH)],
                             rows.at[b], lsem.at[b])
        return _

    lax.fori_loop(0, ngrp - 1, group, 0)
    for b in range(NBUF):
        j = (ngrp - 1) * NBUF + b
        sdst = shared.at[idx2d.at[ioff + j]]
        pltpu.make_async_copy(ne_hbm.at[pl.ds(base + j * CH, CH)],
                              rows.at[b], lsem.at[b]).wait()
        pltpu.async_copy(rows.at[b], sdst, ssem.at[b], add=True)
        pltpu.make_async_copy(rows.at[b], sdst, ssem.at[b]).wait()
    plsc.subcore_barrier()

    pltpu.sync_copy(shared.at[pl.ds(s * ROWS_PER_TILE, ROWS_PER_TILE)], zbuf)
    pltpu.sync_copy(zbuf, out_hbm.at[c, pl.ds(s * ROWS_PER_TILE, ROWS_PER_TILE)])


def kernel(nodes, edges, globals_, senders, receivers,
           We1, be1, We2, be2, Wn1, bn1, Wn2, bn2, Wg1, bg1, Wg2, bg2):
    del globals_
    f32 = jnp.float32
    bf16 = jnp.bfloat16
    i32 = jnp.int32
    recv2d = receivers.astype(i32).reshape(IDXR, CH)
    send2d = senders.astype(i32).reshape(IDXR, CH)

    We1r = We1[D_EDGE:D_EDGE + D_NODE]
    We1s = We1[D_EDGE + D_NODE:]

    pr, ps = pl.pallas_call(
        _prep_body,
        out_shape=[jax.ShapeDtypeStruct((N_NODES, HIDDEN), bf16),
                   jax.ShapeDtypeStruct((N_NODES, HIDDEN), bf16)],
    )(nodes, We1r, We1s)

    mesh = plsc.VectorSubcoreMesh(core_axis_name="c", subcore_axis_name="s",
                                  num_cores=NC, num_subcores=NS)
    gather_k = pl.kernel(
        _sc_gather_body,
        out_type=[jax.ShapeDtypeStruct((N_EDGES, HIDDEN), bf16),
                  jax.ShapeDtypeStruct((N_EDGES, HIDDEN), bf16)],
        mesh=mesh,
        compiler_params=pltpu.CompilerParams(use_tc_tiling_on_sc=False),
        scratch_types=[
            pltpu.VMEM_SHARED((2, N_NODES, HIDDEN), bf16),
            pltpu.VMEM((ROWS_PER_TILE, HIDDEN), bf16),
            pltpu.VMEM((CPW, CH), i32),
            pltpu.VMEM((NBUF, CH, HIDDEN), bf16),
            pltpu.SemaphoreType.DMA((NBUF,)),
            pltpu.SemaphoreType.DMA((NBUF,)),
        ],
    )
    g1, g2 = gather_k(pr, ps, recv2d, send2d)

    edges_t = edges.T
    new_edges_rm, new_edges_t, psum = pl.pallas_call(
        _edge_body,
        grid=(NEBLK,),
        in_specs=[
            pl.BlockSpec((D_EDGE, EBLK), lambda i: (0, i)),
            pl.BlockSpec((EBLK, HIDDEN), lambda i: (i, 0)),
            pl.BlockSpec((EBLK, HIDDEN), lambda i: (i, 0)),
            pl.BlockSpec((D_EDGE, HIDDEN), lambda i: (0, 0)),
            pl.BlockSpec((1, HIDDEN), lambda i: (0, 0)),
            pl.BlockSpec((HIDDEN, HIDDEN), lambda i: (0, 0)),
            pl.BlockSpec((1, HIDDEN), lambda i: (0, 0)),
        ],
        out_specs=[
            pl.BlockSpec((EBLK, HIDDEN), lambda i: (i, 0)),
            pl.BlockSpec((HIDDEN, EBLK), lambda i: (0, i)),
            pl.BlockSpec((1, HIDDEN), lambda i: (0, 0)),
        ],
        out_shape=[jax.ShapeDtypeStruct((N_EDGES, HIDDEN), f32),
                   jax.ShapeDtypeStruct((HIDDEN, N_EDGES), f32),
                   jax.ShapeDtypeStruct((1, HIDDEN), f32)],
    )(edges_t, g1, g2, We1[:D_EDGE], be1[None, :], We2, be2[None, :])
    new_edges = new_edges_t.T

    scatter_k = pl.kernel(
        _sc_scatter_body,
        out_type=jax.ShapeDtypeStruct((NC, N_NODES, HIDDEN), f32),
        mesh=mesh,
        compiler_params=pltpu.CompilerParams(use_tc_tiling_on_sc=False),
        scratch_types=[
            pltpu.VMEM_SHARED((N_NODES, HIDDEN), f32),
            pltpu.VMEM((ROWS_PER_TILE, HIDDEN), f32),
            pltpu.VMEM((CPW, CH), i32),
            pltpu.VMEM((NBUF, CH, HIDDEN), f32),
            pltpu.SemaphoreType.DMA((NBUF,)),
            pltpu.SemaphoreType.DMA((NBUF,)),
        ],
    )
    agg2 = scatter_k(new_edges_rm, recv2d)

    new_nodes, new_globals = pl.pallas_call(
        _node_body,
        out_shape=[jax.ShapeDtypeStruct((N_NODES, HIDDEN), f32),
                   jax.ShapeDtypeStruct((1, HIDDEN), f32)],
    )(nodes, agg2, psum, Wn1[:D_NODE], Wn1[D_NODE:], bn1[None, :], Wn2,
      bn2[None, :], Wg1, bg1[None, :], Wg2, bg2[None, :])

    return (new_nodes, new_edges, new_globals)


# speedup vs baseline: 1.5345x; 1.1477x over previous
import jax
import jax.numpy as jnp
from jax import lax
from jax.experimental import pallas as pl
from jax.experimental.pallas import tpu as pltpu
from jax.experimental.pallas import tpu_sc as plsc

N_NODES = 10000
N_EDGES = 320000
D_NODE = 128
D_EDGE = 16
HIDDEN = 32

NC = 2
NS = 16
NW = NC * NS
CH = 128
CPW = 80
PER_W = CH * CPW
NBUF = 4
LAST_CH = (N_EDGES - (NW - 1) * PER_W) // CH
ROWS_PER_TILE = N_NODES // NS

PACK = 4
ER = N_EDGES // PACK
EBLK = 2000
NEBLK = ER // EBLK

_slope = 0.01


def _leaky(x):
    return jnp.where(x >= 0, x, _slope * x)


def _prep_body(nodes_ref, wr_ref, ws_ref, pr_ref, ps_ref):
    n = nodes_ref[...]
    pr_ref[...] = jnp.dot(
        n, wr_ref[...], preferred_element_type=jnp.float32).astype(jnp.bfloat16)
    ps_ref[...] = jnp.dot(
        n, ws_ref[...], preferred_element_type=jnp.float32).astype(jnp.bfloat16)


def _edge_body(e_ref, g1_ref, g2_ref, w1_ref, b1_ref, w2_ref, b2_ref,
               out_ref, ps_ref):
    i = pl.program_id(0)
    x = jnp.dot(e_ref[...], w1_ref[...], preferred_element_type=jnp.float32)
    g = g1_ref[...].astype(jnp.float32) + g2_ref[...].astype(jnp.float32)
    x = _leaky(x + b1_ref[...] + g)
    y = jnp.dot(x, w2_ref[...], preferred_element_type=jnp.float32)
    y = _leaky(y + b2_ref[...])
    out_ref[...] = y

    @pl.when(i == 0)
    def _():
        ps_ref[...] = jnp.zeros_like(ps_ref)

    ps_ref[...] += jnp.sum(y, axis=0, keepdims=True)


def _node_body(nodes_ref, agg2_ref, psum_ref,
               wn1a_ref, wn1b_ref, bn1_ref, wn2_ref, bn2_ref,
               wg1_ref, bg1_ref, wg2_ref, bg2_ref,
               nn_ref, g_ref):
    agg = agg2_ref[0] + agg2_ref[1]
    h = jnp.dot(nodes_ref[...], wn1a_ref[...], preferred_element_type=jnp.float32)
    h = h + jnp.dot(agg, wn1b_ref[...], preferred_element_type=jnp.float32)
    h = _leaky(h + bn1_ref[...])
    nn = jnp.dot(h, wn2_ref[...], preferred_element_type=jnp.float32)
    nn = _leaky(nn + bn2_ref[...])
    nn_ref[...] = nn

    node_sum = jnp.sum(nn, axis=0, keepdims=True)
    p = psum_ref[...]
    edge_sum = p[:, 0:32] + p[:, 32:64] + p[:, 64:96] + p[:, 96:128]
    gi = jnp.dot(node_sum, wg1_ref[0:HIDDEN, :], preferred_element_type=jnp.float32)
    gi = gi + jnp.dot(edge_sum, wg1_ref[HIDDEN:2 * HIDDEN, :],
                      preferred_element_type=jnp.float32)
    gi = _leaky(gi + bg1_ref[...])
    go = jnp.dot(gi, wg2_ref[...], preferred_element_type=jnp.float32)
    g_ref[...] = _leaky(go + bg2_ref[...])


def _sc_gather_body(pr_hbm, ps_hbm, recv_hbm, send_hbm, g1_hbm, g2_hbm,
                    tabs_sh, tbuf, idx1, rows, gsem, wsem):
    c = lax.axis_index("c")
    s = lax.axis_index("s")
    wid = s * NC + c
    base = wid * PER_W
    nch = jnp.where(wid == NW - 1, LAST_CH, CPW)
    ngrp = nch // NBUF
    off0 = jnp.where(wid == NW - 1, N_EDGES - PER_W, base)
    ioff = base - off0

    for t, tab_hbm in ((0, pr_hbm), (1, ps_hbm)):
        pltpu.sync_copy(tab_hbm.at[pl.ds(s * ROWS_PER_TILE, ROWS_PER_TILE)], tbuf)
        pltpu.sync_copy(tbuf, tabs_sh.at[t, pl.ds(s * ROWS_PER_TILE, ROWS_PER_TILE)])
    plsc.subcore_barrier()

    for t, idx_hbm, out_hbm in ((0, recv_hbm, g1_hbm), (1, send_hbm, g2_hbm)):
        tab = tabs_sh.at[t]
        pltpu.sync_copy(idx_hbm.at[pl.ds(off0, PER_W)], idx1)

        def fire(j, b, tab=tab):
            pltpu.async_copy(tab.at[idx1.at[pl.ds(ioff + j * CH, CH)]],
                             rows.at[b], gsem.at[b])

        def wait_fire(j, b, tab=tab):
            pltpu.make_async_copy(tab.at[idx1.at[pl.ds(ioff + j * CH, CH)]],
                                  rows.at[b], gsem.at[b]).wait()

        for b in range(NBUF):
            fire(b, b)

        def group(g, _, tab=tab, out_hbm=out_hbm):
            for b in range(NBUF):
                j = g * NBUF + b
                dst = out_hbm.at[pl.ds(base + j * CH, CH)]
                wait_fire(j, b)
                pltpu.async_copy(rows.at[b], dst, wsem.at[b])
                pltpu.make_async_copy(rows.at[b], dst, wsem.at[b]).wait()
                fire(NBUF + j, b)
            return _

        lax.fori_loop(0, ngrp - 1, group, 0)
        for b in range(NBUF):
            j = (ngrp - 1) * NBUF + b
            wait_fire(j, b)
            pltpu.async_copy(rows.at[b], out_hbm.at[pl.ds(base + j * CH, CH)],
                             wsem.at[b])
        for b in range(NBUF):
            j = (ngrp - 1) * NBUF + b
            pltpu.make_async_copy(rows.at[b], out_hbm.at[pl.ds(base + j * CH, CH)],
                                  wsem.at[b]).wait()


def _sc_scatter_body(ne_hbm, recv_hbm, out_hbm,
                     shared, zbuf, idxr, rows, lsem, ssem):
    c = lax.axis_index("c")
    s = lax.axis_index("s")
    wid = s * NC + c
    base = wid * PER_W
    nch = jnp.where(wid == NW - 1, LAST_CH, CPW)
    ngrp = nch // NBUF

    def zrow(i, _):
        zbuf[i, pl.ds(0, 16)] = jnp.zeros((16,), jnp.float32)
        zbuf[i, pl.ds(16, 16)] = jnp.zeros((16,), jnp.float32)
        return _

    lax.fori_loop(0, ROWS_PER_TILE, zrow, 0)
    pltpu.sync_copy(zbuf, shared.at[pl.ds(s * ROWS_PER_TILE, ROWS_PER_TILE)])
    plsc.subcore_barrier()

    def fire(j, b):
        pltpu.async_copy(recv_hbm.at[pl.ds(base + j * CH, CH)], idxr.at[b],
                         lsem.at[b])
        pltpu.async_copy(ne_hbm.at[pl.ds(base + j * CH, CH)], rows.at[b],
                         lsem.at[b])

    def wait_loads(j, b):
        pltpu.make_async_copy(recv_hbm.at[pl.ds(base + j * CH, CH)], idxr.at[b],
                              lsem.at[b]).wait()
        pltpu.make_async_copy(ne_hbm.at[pl.ds(base + j * CH, CH)], rows.at[b],
                              lsem.at[b]).wait()

    for b in range(NBUF):
        fire(b, b)

    def group(g, _):
        for b in range(NBUF):
            j = g * NBUF + b
            sdst = shared.at[idxr.at[b]]
            wait_loads(j, b)
            pltpu.async_copy(rows.at[b], sdst, ssem.at[b], add=True)
            pltpu.make_async_copy(rows.at[b], sdst, ssem.at[b]).wait()
            fire(NBUF + j, b)
        return _

    lax.fori_loop(0, ngrp - 1, group, 0)
    for b in range(NBUF):
        j = (ngrp - 1) * NBUF + b
        sdst = shared.at[idxr.at[b]]
        wait_loads(j, b)
        pltpu.async_copy(rows.at[b], sdst, ssem.at[b], add=True)
        pltpu.make_async_copy(rows.at[b], sdst, ssem.at[b]).wait()
    plsc.subcore_barrier()

    pltpu.sync_copy(shared.at[pl.ds(s * ROWS_PER_TILE, ROWS_PER_TILE)], zbuf)
    pltpu.sync_copy(zbuf, out_hbm.at[c, pl.ds(s * ROWS_PER_TILE, ROWS_PER_TILE)])


def kernel(nodes, edges, globals_, senders, receivers,
           We1, be1, We2, be2, Wn1, bn1, Wn2, bn2, Wg1, bg1, Wg2, bg2):
    del globals_
    f32 = jnp.float32
    bf16 = jnp.bfloat16
    recv = receivers.astype(jnp.int32)
    send = senders.astype(jnp.int32)

    We1r = We1[D_EDGE:D_EDGE + D_NODE]
    We1s = We1[D_EDGE + D_NODE:]

    pr, ps = pl.pallas_call(
        _prep_body,
        out_shape=[jax.ShapeDtypeStruct((N_NODES, HIDDEN), bf16),
                   jax.ShapeDtypeStruct((N_NODES, HIDDEN), bf16)],
    )(nodes, We1r, We1s)

    mesh = plsc.VectorSubcoreMesh(core_axis_name="c", subcore_axis_name="s",
                                  num_cores=NC, num_subcores=NS)
    gather_k = pl.kernel(
        _sc_gather_body,
        out_type=[jax.ShapeDtypeStruct((N_EDGES, HIDDEN), bf16),
                  jax.ShapeDtypeStruct((N_EDGES, HIDDEN), bf16)],
        mesh=mesh,
        compiler_params=pltpu.CompilerParams(use_tc_tiling_on_sc=False),
        scratch_types=[
            pltpu.VMEM_SHARED((2, N_NODES, HIDDEN), bf16),
            pltpu.VMEM((ROWS_PER_TILE, HIDDEN), bf16),
            pltpu.VMEM((PER_W,), jnp.int32),
            pltpu.VMEM((NBUF, CH, HIDDEN), bf16),
            pltpu.SemaphoreType.DMA((NBUF,)),
            pltpu.SemaphoreType.DMA((NBUF,)),
        ],
    )
    g1, g2 = gather_k(pr, ps, recv, send)

    W1bd = jax.scipy.linalg.block_diag(*([We1[:D_EDGE]] * PACK))
    W2bd = jax.scipy.linalg.block_diag(*([We2] * PACK))
    b1t = jnp.tile(be1, PACK)[None, :]
    b2t = jnp.tile(be2, PACK)[None, :]
    edges_r = edges.reshape(ER, PACK * D_EDGE)
    g1r = g1.reshape(ER, PACK * HIDDEN)
    g2r = g2.reshape(ER, PACK * HIDDEN)
    new_edges_rp, psum = pl.pallas_call(
        _edge_body,
        grid=(NEBLK,),
        in_specs=[
            pl.BlockSpec((EBLK, PACK * D_EDGE), lambda i: (i, 0)),
            pl.BlockSpec((EBLK, PACK * HIDDEN), lambda i: (i, 0)),
            pl.BlockSpec((EBLK, PACK * HIDDEN), lambda i: (i, 0)),
            pl.BlockSpec((PACK * D_EDGE, PACK * HIDDEN), lambda i: (0, 0)),
            pl.BlockSpec((1, PACK * HIDDEN), lambda i: (0, 0)),
            pl.BlockSpec((PACK * HIDDEN, PACK * HIDDEN), lambda i: (0, 0)),
            pl.BlockSpec((1, PACK * HIDDEN), lambda i: (0, 0)),
        ],
        out_specs=[
            pl.BlockSpec((EBLK, PACK * HIDDEN), lambda i: (i, 0)),
            pl.BlockSpec((1, PACK * HIDDEN), lambda i: (0, 0)),
        ],
        out_shape=[jax.ShapeDtypeStruct((ER, PACK * HIDDEN), f32),
                   jax.ShapeDtypeStruct((1, PACK * HIDDEN), f32)],
    )(edges_r, g1r, g2r, W1bd, b1t, W2bd, b2t)
    new_edges = new_edges_rp.reshape(N_EDGES, HIDDEN)

    scatter_k = pl.kernel(
        _sc_scatter_body,
        out_type=jax.ShapeDtypeStruct((NC, N_NODES, HIDDEN), f32),
        mesh=mesh,
        compiler_params=pltpu.CompilerParams(use_tc_tiling_on_sc=False),
        scratch_types=[
            pltpu.VMEM_SHARED((N_NODES, HIDDEN), f32),
            pltpu.VMEM((ROWS_PER_TILE, HIDDEN), f32),
            pltpu.VMEM((NBUF, CH), jnp.int32),
            pltpu.VMEM((NBUF, CH, HIDDEN), f32),
            pltpu.SemaphoreType.DMA((NBUF,)),
            pltpu.SemaphoreType.DMA((NBUF,)),
        ],
    )
    agg2 = scatter_k(new_edges, recv)

    new_nodes, new_globals = pl.pallas_call(
        _node_body,
        out_shape=[jax.ShapeDtypeStruct((N_NODES, HIDDEN), f32),
                   jax.ShapeDtypeStruct((1, HIDDEN), f32)],
    )(nodes, agg2, psum, Wn1[:D_NODE], Wn1[D_NODE:], bn1[None, :], Wn2,
      bn2[None, :], Wg1, bg1[None, :], Wg2, bg2[None, :])

    return (new_nodes, new_edges, new_globals)


# speedup vs baseline: 2.1575x; 1.4059x over previous
import jax
import jax.numpy as jnp
from jax import lax
from jax.experimental import pallas as pl
from jax.experimental.pallas import tpu as pltpu
from jax.experimental.pallas import tpu_sc as plsc

N_NODES = 10000
N_EDGES = 320000
D_NODE = 128
D_EDGE = 16
HIDDEN = 32

NC = 2
NS = 16
NW = NC * NS
CH = 128
CPW = 80
PER_W = CH * CPW
NBUF = 4
LAST_CH = (N_EDGES - (NW - 1) * PER_W) // CH
ROWS_PER_TILE = N_NODES // NS

PACK = 4
ER = N_EDGES // PACK
EBLK = 2000
NEBLK = ER // EBLK

_slope = 0.01


def _leaky(x):
    return jnp.where(x >= 0, x, _slope * x)


def _prep_body(nodes_ref, wr_ref, ws_ref, pr_ref, ps_ref):
    n = nodes_ref[...]
    pr_ref[...] = jnp.dot(n, wr_ref[...], preferred_element_type=jnp.float32)
    ps_ref[...] = jnp.dot(n, ws_ref[...], preferred_element_type=jnp.float32)


def _edge_body(e_ref, g1_ref, g2_ref, w1_ref, b1_ref, w2_ref, b2_ref,
               out_ref, ps_ref):
    i = pl.program_id(0)
    x = jnp.dot(e_ref[...], w1_ref[...], preferred_element_type=jnp.float32)
    g = g1_ref[...] + g2_ref[...]
    x = _leaky(x + b1_ref[...] + g)
    y = jnp.dot(x, w2_ref[...], preferred_element_type=jnp.float32)
    y = _leaky(y + b2_ref[...])
    out_ref[...] = y

    @pl.when(i == 0)
    def _():
        ps_ref[...] = jnp.zeros_like(ps_ref)

    ps_ref[...] += jnp.sum(y, axis=0, keepdims=True)


def _node_body(nodes_ref, agg2_ref, psum_ref,
               wn1a_ref, wn1b_ref, bn1_ref, wn2_ref, bn2_ref,
               wg1_ref, bg1_ref, wg2_ref, bg2_ref,
               nn_ref, g_ref):
    agg = agg2_ref[0] + agg2_ref[1]
    h = jnp.dot(nodes_ref[...], wn1a_ref[...], preferred_element_type=jnp.float32)
    h = h + jnp.dot(agg, wn1b_ref[...], preferred_element_type=jnp.float32)
    h = _leaky(h + bn1_ref[...])
    nn = jnp.dot(h, wn2_ref[...], preferred_element_type=jnp.float32)
    nn = _leaky(nn + bn2_ref[...])
    nn_ref[...] = nn

    node_sum = jnp.sum(nn, axis=0, keepdims=True)
    p = psum_ref[...]
    edge_sum = p[:, 0:32] + p[:, 32:64] + p[:, 64:96] + p[:, 96:128]
    gi = jnp.dot(node_sum, wg1_ref[0:HIDDEN, :], preferred_element_type=jnp.float32)
    gi = gi + jnp.dot(edge_sum, wg1_ref[HIDDEN:2 * HIDDEN, :],
                      preferred_element_type=jnp.float32)
    gi = _leaky(gi + bg1_ref[...])
    go = jnp.dot(gi, wg2_ref[...], preferred_element_type=jnp.float32)
    g_ref[...] = _leaky(go + bg2_ref[...])


def _sc_gather_body(pr_hbm, ps_hbm, recv_hbm, send_hbm, g1_hbm, g2_hbm,
                    tabs_sh, tbuf, idx1, rows, gsem, wsem):
    c = lax.axis_index("c")
    s = lax.axis_index("s")
    wid = s * NC + c
    base = wid * PER_W
    nch = jnp.where(wid == NW - 1, LAST_CH, CPW)
    ngrp = nch // NBUF
    off0 = jnp.where(wid == NW - 1, N_EDGES - PER_W, base)
    ioff = base - off0

    for t, tab_hbm in ((0, pr_hbm), (1, ps_hbm)):
        pltpu.sync_copy(tab_hbm.at[pl.ds(s * ROWS_PER_TILE, ROWS_PER_TILE)], tbuf)
        pltpu.sync_copy(tbuf, tabs_sh.at[t, pl.ds(s * ROWS_PER_TILE, ROWS_PER_TILE)])
    plsc.subcore_barrier()

    for t, idx_hbm, out_hbm in ((0, recv_hbm, g1_hbm), (1, send_hbm, g2_hbm)):
        tab = tabs_sh.at[t]
        pltpu.sync_copy(idx_hbm.at[pl.ds(off0, PER_W)], idx1)

        def fire(j, b, tab=tab):
            pltpu.async_copy(tab.at[idx1.at[pl.ds(ioff + j * CH, CH)]],
                             rows.at[b], gsem.at[b])

        def wait_fire(j, b, tab=tab):
            pltpu.make_async_copy(tab.at[idx1.at[pl.ds(ioff + j * CH, CH)]],
                                  rows.at[b], gsem.at[b]).wait()

        for b in range(NBUF):
            fire(b, b)

        def group(g, _, tab=tab, out_hbm=out_hbm):
            for b in range(NBUF):
                j = g * NBUF + b
                dst = out_hbm.at[pl.ds(base + j * CH, CH)]
                wait_fire(j, b)
                pltpu.async_copy(rows.at[b], dst, wsem.at[b])
                pltpu.make_async_copy(rows.at[b], dst, wsem.at[b]).wait()
                fire(NBUF + j, b)
            return _

        lax.fori_loop(0, ngrp - 1, group, 0)
        for b in range(NBUF):
            j = (ngrp - 1) * NBUF + b
            wait_fire(j, b)
            pltpu.async_copy(rows.at[b], out_hbm.at[pl.ds(base + j * CH, CH)],
                             wsem.at[b])
        for b in range(NBUF):
            j = (ngrp - 1) * NBUF + b
            pltpu.make_async_copy(rows.at[b], out_hbm.at[pl.ds(base + j * CH, CH)],
                                  wsem.at[b]).wait()


def _sc_scatter_body(ne_hbm, recv_hbm, out_hbm,
                     shared, zbuf, idxr, rows, lsem, ssem):
    c = lax.axis_index("c")
    s = lax.axis_index("s")
    wid = s * NC + c
    base = wid * PER_W
    nch = jnp.where(wid == NW - 1, LAST_CH, CPW)
    ngrp = nch // NBUF

    def zrow(i, _):
        zbuf[i, pl.ds(0, 16)] = jnp.zeros((16,), jnp.float32)
        zbuf[i, pl.ds(16, 16)] = jnp.zeros((16,), jnp.float32)
        return _

    lax.fori_loop(0, ROWS_PER_TILE, zrow, 0)
    pltpu.sync_copy(zbuf, shared.at[pl.ds(s * ROWS_PER_TILE, ROWS_PER_TILE)])
    plsc.subcore_barrier()

    def fire(j, b):
        pltpu.async_copy(recv_hbm.at[pl.ds(base + j * CH, CH)], idxr.at[b],
                         lsem.at[b])
        pltpu.async_copy(ne_hbm.at[pl.ds(base + j * CH, CH)], rows.at[b],
                         lsem.at[b])

    def wait_loads(j, b):
        pltpu.make_async_copy(recv_hbm.at[pl.ds(base + j * CH, CH)], idxr.at[b],
                              lsem.at[b]).wait()
        pltpu.make_async_copy(ne_hbm.at[pl.ds(base + j * CH, CH)], rows.at[b],
                              lsem.at[b]).wait()

    for b in range(NBUF):
        fire(b, b)

    def group(g, _):
        for b in range(NBUF):
            j = g * NBUF + b
            sdst = shared.at[idxr.at[b]]
            wait_loads(j, b)
            pltpu.async_copy(rows.at[b], sdst, ssem.at[b], add=True)
            pltpu.make_async_copy(rows.at[b], sdst, ssem.at[b]).wait()
            fire(NBUF + j, b)
        return _

    lax.fori_loop(0, ngrp - 1, group, 0)
    for b in range(NBUF):
        j = (ngrp - 1) * NBUF + b
        sdst = shared.at[idxr.at[b]]
        wait_loads(j, b)
        pltpu.async_copy(rows.at[b], sdst, ssem.at[b], add=True)
        pltpu.make_async_copy(rows.at[b], sdst, ssem.at[b]).wait()
    plsc.subcore_barrier()

    pltpu.sync_copy(shared.at[pl.ds(s * ROWS_PER_TILE, ROWS_PER_TILE)], zbuf)
    pltpu.sync_copy(zbuf, out_hbm.at[c, pl.ds(s * ROWS_PER_TILE, ROWS_PER_TILE)])


def kernel(nodes, edges, globals_, senders, receivers,
           We1, be1, We2, be2, Wn1, bn1, Wn2, bn2, Wg1, bg1, Wg2, bg2):
    del globals_
    f32 = jnp.float32
    recv = receivers.astype(jnp.int32)
    send = senders.astype(jnp.int32)

    We1r = We1[D_EDGE:D_EDGE + D_NODE]
    We1s = We1[D_EDGE + D_NODE:]

    pr, ps = pl.pallas_call(
        _prep_body,
        out_shape=[jax.ShapeDtypeStruct((N_NODES, HIDDEN), f32),
                   jax.ShapeDtypeStruct((N_NODES, HIDDEN), f32)],
    )(nodes, We1r, We1s)

    mesh = plsc.VectorSubcoreMesh(core_axis_name="c", subcore_axis_name="s",
                                  num_cores=NC, num_subcores=NS)
    gather_k = pl.kernel(
        _sc_gather_body,
        out_type=[jax.ShapeDtypeStruct((N_EDGES, HIDDEN), f32),
                  jax.ShapeDtypeStruct((N_EDGES, HIDDEN), f32)],
        mesh=mesh,
        compiler_params=pltpu.CompilerParams(use_tc_tiling_on_sc=False),
        scratch_types=[
            pltpu.VMEM_SHARED((2, N_NODES, HIDDEN), f32),
            pltpu.VMEM((ROWS_PER_TILE, HIDDEN), f32),
            pltpu.VMEM((PER_W,), jnp.int32),
            pltpu.VMEM((NBUF, CH, HIDDEN), f32),
            pltpu.SemaphoreType.DMA((NBUF,)),
            pltpu.SemaphoreType.DMA((NBUF,)),
        ],
    )
    g1, g2 = gather_k(pr, ps, recv, send)

    W1bd = jax.scipy.linalg.block_diag(*([We1[:D_EDGE]] * PACK))
    W2bd = jax.scipy.linalg.block_diag(*([We2] * PACK))
    b1t = jnp.tile(be1, PACK)[None, :]
    b2t = jnp.tile(be2, PACK)[None, :]
    edges_r = edges.reshape(ER, PACK * D_EDGE)
    g1r = g1.reshape(ER, PACK * HIDDEN)
    g2r = g2.reshape(ER, PACK * HIDDEN)
    new_edges_rp, psum = pl.pallas_call(
        _edge_body,
        grid=(NEBLK,),
        in_specs=[
            pl.BlockSpec((EBLK, PACK * D_EDGE), lambda i: (i, 0)),
            pl.BlockSpec((EBLK, PACK * HIDDEN), lambda i: (i, 0)),
            pl.BlockSpec((EBLK, PACK * HIDDEN), lambda i: (i, 0)),
            pl.BlockSpec((PACK * D_EDGE, PACK * HIDDEN), lambda i: (0, 0)),
            pl.BlockSpec((1, PACK * HIDDEN), lambda i: (0, 0)),
            pl.BlockSpec((PACK * HIDDEN, PACK * HIDDEN), lambda i: (0, 0)),
            pl.BlockSpec((1, PACK * HIDDEN), lambda i: (0, 0)),
        ],
        out_specs=[
            pl.BlockSpec((EBLK, PACK * HIDDEN), lambda i: (i, 0)),
            pl.BlockSpec((1, PACK * HIDDEN), lambda i: (0, 0)),
        ],
        out_shape=[jax.ShapeDtypeStruct((ER, PACK * HIDDEN), f32),
                   jax.ShapeDtypeStruct((1, PACK * HIDDEN), f32)],
    )(edges_r, g1r, g2r, W1bd, b1t, W2bd, b2t)
    new_edges = new_edges_rp.reshape(N_EDGES, HIDDEN)

    scatter_k = pl.kernel(
        _sc_scatter_body,
        out_type=jax.ShapeDtypeStruct((NC, N_NODES, HIDDEN), f32),
        mesh=mesh,
        compiler_params=pltpu.CompilerParams(use_tc_tiling_on_sc=False),
        scratch_types=[
            pltpu.VMEM_SHARED((N_NODES, HIDDEN), f32),
            pltpu.VMEM((ROWS_PER_TILE, HIDDEN), f32),
            pltpu.VMEM((NBUF, CH), jnp.int32),
            pltpu.VMEM((NBUF, CH, HIDDEN), f32),
            pltpu.SemaphoreType.DMA((NBUF,)),
            pltpu.SemaphoreType.DMA((NBUF,)),
        ],
    )
    agg2 = scatter_k(new_edges, recv)

    new_nodes, new_globals = pl.pallas_call(
        _node_body,
        out_shape=[jax.ShapeDtypeStruct((N_NODES, HIDDEN), f32),
                   jax.ShapeDtypeStruct((1, HIDDEN), f32)],
    )(nodes, agg2, psum, Wn1[:D_NODE], Wn1[D_NODE:], bn1[None, :], Wn2,
      bn2[None, :], Wg1, bg1[None, :], Wg2, bg2[None, :])

    return (new_nodes, new_edges, new_globals)
